# Initial kernel scaffold; baseline (speedup 1.0000x reference)
import jax
import jax.numpy as jnp
from jax.experimental import pallas as pl


def kernel(x_0, table_0, table_1, table_2, table_3, table_4, table_5, table_6, table_7, table_8):
    raise NotImplementedError("write your pallas kernel here")


# speedup vs baseline: 7.4227x; 7.4227x over previous
import functools

import jax
import jax.numpy as jnp
from jax.experimental import pallas as pl
from jax.experimental.pallas import tpu as pltpu

_DIMS = [119, 5, 12, 12, 10, 6, 6, 2, 2]
_OFF = [0, 119, 124, 136, 148, 158, 164, 170, 172]
_TOTP = 176
_EMB = 128
_B = 1024


def _body(x_ref, tbl_ref, o_ref):
    x = x_ref[...]
    r = jax.lax.broadcasted_iota(jnp.int32, (_B, _TOTP), 1)
    mh = jnp.zeros((_B, _TOTP), jnp.float32)
    for i in range(9):
        c = x[:, i][:, None] + _OFF[i]
        mh = mh + (c == r).astype(jnp.float32)
    o_ref[...] = jnp.dot(mh, tbl_ref[...], preferred_element_type=jnp.float32)


@jax.jit
def kernel(x_0, table_0, table_1, table_2, table_3, table_4, table_5,
           table_6, table_7, table_8):
    n = x_0.shape[0]
    tables = [table_0, table_1, table_2, table_3, table_4, table_5,
              table_6, table_7, table_8]
    stacked = jnp.concatenate(
        tables + [jnp.zeros((_TOTP - sum(_DIMS), _EMB), jnp.float32)], axis=0)
    npad = ((n + _B - 1) // _B) * _B
    xp = jnp.pad(x_0, ((0, npad - n), (0, 0)))
    grid = (npad // _B,)
    out = pl.pallas_call(
        _body,
        grid=grid,
        in_specs=[
            pl.BlockSpec((_B, 9), lambda i: (i, 0)),
            pl.BlockSpec((_TOTP, _EMB), lambda i: (0, 0)),
        ],
        out_specs=pl.BlockSpec((_B, _EMB), lambda i: (i, 0)),
        out_shape=jax.ShapeDtypeStruct((npad, _EMB), jnp.float32),
    )(xp, stacked)
    return out[:n]
